# baseline (speedup 1.0000x reference)
import jax
import jax.numpy as jnp
from jax import lax
from jax.experimental import pallas as pl
from jax.experimental.pallas import tpu as pltpu
from jax.experimental.pallas import tpu_sc as plsc

N = 10000
NP = 10112
D = 128
E = 320000
K = 128
NTILES = 32
CH = 80
CH2 = CH // 2
E_PAD = NTILES * CH * K
RPT = NP // 16

_mesh = plsc.VectorSubcoreMesh(core_axis_name="c", subcore_axis_name="s")


def _sc_deg_body(dst_hbm, ones_hbm, zeros_hbm, out_hbm, idx_v, ones_v, tmp, acc):
    c = lax.axis_index("c")
    s = lax.axis_index("s")
    r0 = s * RPT

    @pl.when(c == 0)
    def _():
        pltpu.sync_copy(ones_hbm.at[pl.ds(r0, RPT)], tmp.at[pl.ds(0, RPT)])

    @pl.when(c != 0)
    def _():
        pltpu.sync_copy(zeros_hbm.at[pl.ds(r0, RPT)], tmp.at[pl.ds(0, RPT)])

    pltpu.sync_copy(tmp.at[pl.ds(0, RPT)], acc.at[pl.ds(r0, RPT)])

    pltpu.sync_copy(ones_hbm.at[pl.ds(0, K)], ones_v)
    wid = c * 16 + s
    pltpu.sync_copy(dst_hbm.at[wid], idx_v)
    plsc.subcore_barrier()

    def body(i, carry):
        pltpu.sync_copy(ones_v, acc.at[idx_v.at[i]], add=True)
        return carry

    lax.fori_loop(0, CH, body, 0)
    plsc.subcore_barrier()

    @pl.when(s == 0)
    def _():
        pltpu.sync_copy(acc, tmp)
        pltpu.sync_copy(tmp, out_hbm.at[c])


_sc_deg = pl.kernel(
    _sc_deg_body,
    out_type=jax.ShapeDtypeStruct((2, NP), jnp.float32),
    mesh=_mesh,
    scratch_types=[
        pltpu.VMEM((CH, K), jnp.int32),
        pltpu.VMEM((K,), jnp.float32),
        pltpu.VMEM((NP,), jnp.float32),
        pltpu.VMEM_SHARED((NP,), jnp.float32),
    ],
)


def _sc_agg_body(hs_hbm, src_hbm, dst_hbm, zeros_hbm, out_hbm,
                 sidx, didx, rows0, rows1, acc, g0, g1):
    c = lax.axis_index("c")
    s = lax.axis_index("s")
    r0 = s * RPT

    @pl.when(c == 0)
    def _():
        pltpu.sync_copy(hs_hbm.at[pl.ds(r0, RPT)], acc.at[pl.ds(r0, RPT)])

    @pl.when(c != 0)
    def _():
        pltpu.sync_copy(zeros_hbm.at[pl.ds(r0, RPT)], acc.at[pl.ds(r0, RPT)])

    wid = c * 16 + s
    plsc.subcore_barrier()

    for p in range(2):
        pltpu.sync_copy(src_hbm.at[wid, pl.ds(p * CH2, CH2)], sidx)
        pltpu.sync_copy(dst_hbm.at[wid, pl.ds(p * CH2, CH2)], didx)
        pltpu.async_copy(hs_hbm.at[sidx.at[0]], rows0, g0)
        pltpu.async_copy(hs_hbm.at[sidx.at[1]], rows1, g1)

        def body(j, carry):
            i0 = 2 * j
            i1 = 2 * j + 1
            pltpu.make_async_copy(hs_hbm.at[sidx.at[i0]], rows0, g0).wait()
            pltpu.sync_copy(rows0, acc.at[didx.at[i0]], add=True)

            @pl.when(i0 + 2 < CH2)
            def _():
                pltpu.async_copy(hs_hbm.at[sidx.at[i0 + 2]], rows0, g0)

            pltpu.make_async_copy(hs_hbm.at[sidx.at[i1]], rows1, g1).wait()
            pltpu.sync_copy(rows1, acc.at[didx.at[i1]], add=True)

            @pl.when(i1 + 2 < CH2)
            def _():
                pltpu.async_copy(hs_hbm.at[sidx.at[i1 + 2]], rows1, g1)

            return carry

        lax.fori_loop(0, CH2 // 2, body, 0)

    plsc.subcore_barrier()
    pltpu.sync_copy(acc.at[pl.ds(r0, RPT)], out_hbm.at[c, pl.ds(r0, RPT)])


_sc_agg = pl.kernel(
    _sc_agg_body,
    out_type=jax.ShapeDtypeStruct((2, NP, D), jnp.float32),
    mesh=_mesh,
    scratch_types=[
        pltpu.VMEM((CH2, K), jnp.int32),
        pltpu.VMEM((CH2, K), jnp.int32),
        pltpu.VMEM((K, D), jnp.float32),
        pltpu.VMEM((K, D), jnp.float32),
        pltpu.VMEM_SHARED((NP, D), jnp.float32),
        pltpu.SemaphoreType.DMA,
        pltpu.SemaphoreType.DMA,
    ],
)


def _leaky(x):
    return jnp.where(x >= 0, x, 0.01 * x)


def _tc_first_body(x_ref, w_ref, deg_ref, hs_ref, dinv_ref):
    deg2 = deg_ref[...]
    deg = (deg2[0] + deg2[1])[:, None]
    row = lax.broadcasted_iota(jnp.int32, (NP, 1), 0)
    dinv = jnp.where((deg > 0) & (row < N), lax.rsqrt(deg), 0.0)
    h = jnp.dot(x_ref[...], w_ref[...], preferred_element_type=jnp.float32)
    hs_ref[...] = h * dinv
    dinv_ref[...] = dinv


def _tc_mid_body(pp_ref, dinv_ref, b_ref, w_ref, hs_ref):
    pp = pp_ref[...]
    dinv = dinv_ref[...]
    x = _leaky(dinv * (pp[0] + pp[1]) + b_ref[...])
    h = jnp.dot(x, w_ref[...], preferred_element_type=jnp.float32)
    hs_ref[...] = h * dinv


def _tc_last_body(pp_ref, dinv_ref, b_ref, out_ref):
    pp = pp_ref[...]
    dinv = dinv_ref[...]
    out_ref[...] = _leaky(dinv * (pp[0] + pp[1]) + b_ref[...])


def kernel(nodes_feature, edge_index, W1, b1, W2, b2):
    f32 = jnp.float32
    src = edge_index[0]
    dst = edge_index[1]
    pad = jnp.full((E_PAD - E,), N, dtype=jnp.int32)
    src_p = jnp.concatenate([src, pad]).reshape(NTILES, CH, K)
    dst_p = jnp.concatenate([dst, pad]).reshape(NTILES, CH, K)
    x_p = jnp.zeros((NP, D), f32).at[:N].set(nodes_feature)
    zeros_nodes = jnp.zeros((NP, D), f32)
    ones_deg = jnp.ones((NP,), f32)
    zeros_deg = jnp.zeros((NP,), f32)
    b1r = b1.reshape(1, D)
    b2r = b2.reshape(1, D)

    deg2 = _sc_deg(dst_p, ones_deg, zeros_deg)

    hs1, dinv = pl.pallas_call(
        _tc_first_body,
        out_shape=[
            jax.ShapeDtypeStruct((NP, D), f32),
            jax.ShapeDtypeStruct((NP, 1), f32),
        ],
    )(x_p, W1, deg2)

    pp1 = _sc_agg(hs1, src_p, dst_p, zeros_nodes)

    hs2 = pl.pallas_call(
        _tc_mid_body,
        out_shape=jax.ShapeDtypeStruct((NP, D), f32),
    )(pp1, dinv, b1r, W2)

    pp2 = _sc_agg(hs2, src_p, dst_p, zeros_nodes)

    out_p = pl.pallas_call(
        _tc_last_body,
        out_shape=jax.ShapeDtypeStruct((NP, D), f32),
    )(pp2, dinv, b2r)

    return out_p[:N]


# speedup vs baseline: 9.9198x; 9.9198x over previous
import jax
import jax.numpy as jnp
from jax import lax
from jax.experimental import pallas as pl
from jax.experimental.pallas import tpu as pltpu
from jax.experimental.pallas import tpu_sc as plsc

N = 10000
NP = 10112
D = 128
E = 320000
K = 128
NTILES = 32
PH = 40
CH0 = 3 * PH
CH1 = PH
NCH = 16 * (CH0 + CH1)
DCH = NCH // NTILES
E_PAD = NCH * K
RPT = NP // 16

_mesh = plsc.VectorSubcoreMesh(core_axis_name="c", subcore_axis_name="s")


def _sc_deg_body(dst_hbm, ones_hbm, zeros_hbm, out_hbm, idx_v, ones_v, tmp, acc):
    c = lax.axis_index("c")
    s = lax.axis_index("s")
    r0 = s * RPT

    @pl.when(c == 0)
    def _():
        pltpu.sync_copy(ones_hbm.at[pl.ds(r0, RPT)], tmp.at[pl.ds(0, RPT)])

    @pl.when(c != 0)
    def _():
        pltpu.sync_copy(zeros_hbm.at[pl.ds(r0, RPT)], tmp.at[pl.ds(0, RPT)])

    pltpu.sync_copy(tmp.at[pl.ds(0, RPT)], acc.at[pl.ds(r0, RPT)])

    pltpu.sync_copy(ones_hbm.at[pl.ds(0, K)], ones_v)
    wid = c * 16 + s
    pltpu.sync_copy(dst_hbm.at[pl.ds(wid * DCH, DCH)], idx_v)
    plsc.subcore_barrier()

    def body(i, carry):
        pltpu.sync_copy(ones_v, acc.at[idx_v.at[i]], add=True)
        return carry

    lax.fori_loop(0, DCH, body, 0)
    plsc.subcore_barrier()

    @pl.when(s == 0)
    def _():
        pltpu.sync_copy(acc, tmp)
        pltpu.sync_copy(tmp, out_hbm.at[c])


_sc_deg = pl.kernel(
    _sc_deg_body,
    out_type=jax.ShapeDtypeStruct((2, NP), jnp.float32),
    mesh=_mesh,
    scratch_types=[
        pltpu.VMEM((DCH, K), jnp.int32),
        pltpu.VMEM((K,), jnp.float32),
        pltpu.VMEM((NP,), jnp.float32),
        pltpu.VMEM_SHARED((NP,), jnp.float32),
    ],
)


def _sc_agg_body(hs_hbm, src_hbm, dst_hbm, zeros_hbm, out_hbm,
                 sidx, didx, rows0, rows1, acc, g0, g1):
    c = lax.axis_index("c")
    s = lax.axis_index("s")
    r0 = s * RPT

    @pl.when(c == 0)
    def _():
        pltpu.sync_copy(hs_hbm.at[pl.ds(r0, RPT)], acc.at[pl.ds(r0, RPT)])

    @pl.when(c != 0)
    def _():
        pltpu.sync_copy(zeros_hbm.at[pl.ds(r0, RPT)], acc.at[pl.ds(r0, RPT)])

    plsc.subcore_barrier()

    def run_phase(base):
        pltpu.sync_copy(src_hbm.at[pl.ds(base, PH)], sidx)
        pltpu.sync_copy(dst_hbm.at[pl.ds(base, PH)], didx)
        pltpu.async_copy(hs_hbm.at[sidx.at[0]], rows0, g0)
        pltpu.async_copy(hs_hbm.at[sidx.at[1]], rows1, g1)

        def body(j, carry):
            i0 = 2 * j
            i1 = 2 * j + 1
            pltpu.make_async_copy(hs_hbm.at[sidx.at[i0]], rows0, g0).wait()
            pltpu.sync_copy(rows0, acc.at[didx.at[i0]], add=True)

            @pl.when(i0 + 2 < PH)
            def _():
                pltpu.async_copy(hs_hbm.at[sidx.at[i0 + 2]], rows0, g0)

            pltpu.make_async_copy(hs_hbm.at[sidx.at[i1]], rows1, g1).wait()
            pltpu.sync_copy(rows1, acc.at[didx.at[i1]], add=True)

            @pl.when(i1 + 2 < PH)
            def _():
                pltpu.async_copy(hs_hbm.at[sidx.at[i1 + 2]], rows1, g1)

            return carry

        lax.fori_loop(0, PH // 2, body, 0)

    @pl.when(c == 0)
    def _():
        for p in range(CH0 // PH):
            run_phase(s * CH0 + p * PH)

    @pl.when(c != 0)
    def _():
        for p in range(CH1 // PH):
            run_phase(16 * CH0 + s * CH1 + p * PH)

    plsc.subcore_barrier()
    pltpu.sync_copy(acc.at[pl.ds(r0, RPT)], out_hbm.at[c, pl.ds(r0, RPT)])


_sc_agg = pl.kernel(
    _sc_agg_body,
    out_type=jax.ShapeDtypeStruct((2, NP, D), jnp.float32),
    mesh=_mesh,
    scratch_types=[
        pltpu.VMEM((PH, K), jnp.int32),
        pltpu.VMEM((PH, K), jnp.int32),
        pltpu.VMEM((K, D), jnp.float32),
        pltpu.VMEM((K, D), jnp.float32),
        pltpu.VMEM_SHARED((NP, D), jnp.float32),
        pltpu.SemaphoreType.DMA,
        pltpu.SemaphoreType.DMA,
    ],
)


def _leaky(x):
    return jnp.where(x >= 0, x, 0.01 * x)


def _tc_first_body(x_ref, w_ref, deg_ref, hs_ref, dinv_ref):
    deg2 = deg_ref[...]
    deg = (deg2[0] + deg2[1])[:, None]
    row = lax.broadcasted_iota(jnp.int32, (NP, 1), 0)
    dinv = jnp.where((deg > 0) & (row < N), lax.rsqrt(deg), 0.0)
    h = jnp.dot(x_ref[...], w_ref[...], preferred_element_type=jnp.float32)
    hs_ref[...] = h * dinv
    dinv_ref[...] = dinv


def _tc_mid_body(pp_ref, dinv_ref, b_ref, w_ref, hs_ref):
    pp = pp_ref[...]
    dinv = dinv_ref[...]
    x = _leaky(dinv * (pp[0] + pp[1]) + b_ref[...])
    h = jnp.dot(x, w_ref[...], preferred_element_type=jnp.float32)
    hs_ref[...] = h * dinv


def _tc_last_body(pp_ref, dinv_ref, b_ref, out_ref):
    pp = pp_ref[...]
    dinv = dinv_ref[...]
    out_ref[...] = _leaky(dinv * (pp[0] + pp[1]) + b_ref[...])


def kernel(nodes_feature, edge_index, W1, b1, W2, b2):
    f32 = jnp.float32
    src = edge_index[0]
    dst = edge_index[1]
    pad = jnp.full((E_PAD - E,), N, dtype=jnp.int32)
    src_p = jnp.concatenate([src, pad]).reshape(NCH, K)
    dst_p = jnp.concatenate([dst, pad]).reshape(NCH, K)
    x_p = jnp.zeros((NP, D), f32).at[:N].set(nodes_feature)
    zeros_nodes = jnp.zeros((NP, D), f32)
    ones_deg = jnp.ones((NP,), f32)
    zeros_deg = jnp.zeros((NP,), f32)
    b1r = b1.reshape(1, D)
    b2r = b2.reshape(1, D)

    deg2 = _sc_deg(dst_p, ones_deg, zeros_deg)

    hs1, dinv = pl.pallas_call(
        _tc_first_body,
        out_shape=[
            jax.ShapeDtypeStruct((NP, D), f32),
            jax.ShapeDtypeStruct((NP, 1), f32),
        ],
    )(x_p, W1, deg2)

    pp1 = _sc_agg(hs1, src_p, dst_p, zeros_nodes)

    hs2 = pl.pallas_call(
        _tc_mid_body,
        out_shape=jax.ShapeDtypeStruct((NP, D), f32),
    )(pp1, dinv, b1r, W2)

    pp2 = _sc_agg(hs2, src_p, dst_p, zeros_nodes)

    out_p = pl.pallas_call(
        _tc_last_body,
        out_shape=jax.ShapeDtypeStruct((NP, D), f32),
    )(pp2, dinv, b2r)

    return out_p[:N]


# speedup vs baseline: 33.0070x; 3.3274x over previous
import jax
import jax.numpy as jnp
from jax import lax
from jax.experimental import pallas as pl
from jax.experimental.pallas import tpu as pltpu
from jax.experimental.pallas import tpu_sc as plsc

N = 10000
NP = 10112
D = 128
E = 320000
K = 128
NTILES = 32
PH = 40
CH = 2 * PH
NCH = NTILES * CH
DCH = NCH // NTILES
E_PAD = NCH * K
RPT = NP // 16

_mesh = plsc.VectorSubcoreMesh(core_axis_name="c", subcore_axis_name="s")


def _sc_deg_body(dst_hbm, ones_hbm, zeros_hbm, out_hbm, idx_v, ones_v, tmp, acc):
    c = lax.axis_index("c")
    s = lax.axis_index("s")
    r0 = s * RPT

    @pl.when(c == 0)
    def _():
        pltpu.sync_copy(ones_hbm.at[pl.ds(r0, RPT)], tmp.at[pl.ds(0, RPT)])

    @pl.when(c != 0)
    def _():
        pltpu.sync_copy(zeros_hbm.at[pl.ds(r0, RPT)], tmp.at[pl.ds(0, RPT)])

    pltpu.sync_copy(tmp.at[pl.ds(0, RPT)], acc.at[pl.ds(r0, RPT)])

    pltpu.sync_copy(ones_hbm.at[pl.ds(0, K)], ones_v)
    wid = c * 16 + s
    pltpu.sync_copy(dst_hbm.at[pl.ds(wid * DCH, DCH)], idx_v)
    plsc.subcore_barrier()

    def body(i, carry):
        pltpu.sync_copy(ones_v, acc.at[idx_v.at[i]], add=True)
        return carry

    lax.fori_loop(0, DCH, body, 0)
    plsc.subcore_barrier()

    @pl.when(s == 0)
    def _():
        pltpu.sync_copy(acc, tmp)
        pltpu.sync_copy(tmp, out_hbm.at[c])


_sc_deg = pl.kernel(
    _sc_deg_body,
    out_type=jax.ShapeDtypeStruct((2, NP), jnp.float32),
    mesh=_mesh,
    scratch_types=[
        pltpu.VMEM((DCH, K), jnp.int32),
        pltpu.VMEM((K,), jnp.float32),
        pltpu.VMEM((NP,), jnp.float32),
        pltpu.VMEM_SHARED((NP,), jnp.float32),
    ],
)


def _sc_agg_body(hs_hbm, src_hbm, dst_hbm, zeros_hbm, out_hbm,
                 sidx, didx, rows0, rows1, acc, g0, g1):
    c = lax.axis_index("c")
    s = lax.axis_index("s")
    r0 = s * RPT

    @pl.when(c == 0)
    def _():
        pltpu.sync_copy(hs_hbm.at[pl.ds(r0, RPT)], acc.at[pl.ds(r0, RPT)])

    @pl.when(c != 0)
    def _():
        pltpu.sync_copy(zeros_hbm.at[pl.ds(r0, RPT)], acc.at[pl.ds(r0, RPT)])

    plsc.subcore_barrier()

    def run_phase(base):
        pltpu.sync_copy(src_hbm.at[pl.ds(base, PH)], sidx)
        pltpu.sync_copy(dst_hbm.at[pl.ds(base, PH)], didx)
        pltpu.async_copy(hs_hbm.at[sidx.at[0]], rows0, g0)
        pltpu.async_copy(hs_hbm.at[sidx.at[1]], rows1, g1)

        def body(j, carry):
            i0 = 2 * j
            i1 = 2 * j + 1
            pltpu.make_async_copy(hs_hbm.at[sidx.at[i0]], rows0, g0).wait()
            pltpu.sync_copy(rows0, acc.at[didx.at[i0]], add=True)

            @pl.when(i0 + 2 < PH)
            def _():
                pltpu.async_copy(hs_hbm.at[sidx.at[i0 + 2]], rows0, g0)

            pltpu.make_async_copy(hs_hbm.at[sidx.at[i1]], rows1, g1).wait()
            pltpu.sync_copy(rows1, acc.at[didx.at[i1]], add=True)

            @pl.when(i1 + 2 < PH)
            def _():
                pltpu.async_copy(hs_hbm.at[sidx.at[i1 + 2]], rows1, g1)

            return carry

        lax.fori_loop(0, PH // 2, body, 0)

    wid = c * 16 + s
    for p in range(CH // PH):
        run_phase(wid * CH + p * PH)

    plsc.subcore_barrier()
    pltpu.sync_copy(acc.at[pl.ds(r0, RPT)], out_hbm.at[c, pl.ds(r0, RPT)])


_sc_agg = pl.kernel(
    _sc_agg_body,
    out_type=jax.ShapeDtypeStruct((2, NP, D), jnp.float32),
    mesh=_mesh,
    scratch_types=[
        pltpu.VMEM((PH, K), jnp.int32),
        pltpu.VMEM((PH, K), jnp.int32),
        pltpu.VMEM((K, D), jnp.float32),
        pltpu.VMEM((K, D), jnp.float32),
        pltpu.VMEM_SHARED((NP, D), jnp.float32),
        pltpu.SemaphoreType.DMA,
        pltpu.SemaphoreType.DMA,
    ],
)


def _leaky(x):
    return jnp.where(x >= 0, x, 0.01 * x)


def _tc_first_body(x_ref, w_ref, deg_ref, hs_ref, dinv_ref):
    deg2 = deg_ref[...]
    deg = (deg2[0] + deg2[1])[:, None]
    row = lax.broadcasted_iota(jnp.int32, (NP, 1), 0)
    dinv = jnp.where((deg > 0) & (row < N), lax.rsqrt(deg), 0.0)
    h = jnp.dot(x_ref[...], w_ref[...], preferred_element_type=jnp.float32)
    hs_ref[...] = h * dinv
    dinv_ref[...] = dinv


def _tc_mid_body(pp_ref, dinv_ref, b_ref, w_ref, hs_ref):
    pp = pp_ref[...]
    dinv = dinv_ref[...]
    x = _leaky(dinv * (pp[0] + pp[1]) + b_ref[...])
    h = jnp.dot(x, w_ref[...], preferred_element_type=jnp.float32)
    hs_ref[...] = h * dinv


def _tc_last_body(pp_ref, dinv_ref, b_ref, out_ref):
    pp = pp_ref[...]
    dinv = dinv_ref[...]
    out_ref[...] = _leaky(dinv * (pp[0] + pp[1]) + b_ref[...])


def kernel(nodes_feature, edge_index, W1, b1, W2, b2):
    f32 = jnp.float32
    src = edge_index[0]
    dst = edge_index[1]
    npad = E_PAD - E
    pad_src = (jnp.arange(npad, dtype=jnp.int32) * 131) % N
    pad_dst = N + (jnp.arange(npad, dtype=jnp.int32) % (NP - N))
    src_p = jnp.concatenate([src, pad_src]).reshape(NCH, K)
    dst_p = jnp.concatenate([dst, pad_dst]).reshape(NCH, K)
    x_p = jnp.zeros((NP, D), f32).at[:N].set(nodes_feature)
    zeros_nodes = jnp.zeros((NP, D), f32)
    ones_deg = jnp.ones((NP,), f32)
    zeros_deg = jnp.zeros((NP,), f32)
    b1r = b1.reshape(1, D)
    b2r = b2.reshape(1, D)

    deg2 = _sc_deg(dst_p, ones_deg, zeros_deg)

    hs1, dinv = pl.pallas_call(
        _tc_first_body,
        out_shape=[
            jax.ShapeDtypeStruct((NP, D), f32),
            jax.ShapeDtypeStruct((NP, 1), f32),
        ],
    )(x_p, W1, deg2)

    pp1 = _sc_agg(hs1, src_p, dst_p, zeros_nodes)

    hs2 = pl.pallas_call(
        _tc_mid_body,
        out_shape=jax.ShapeDtypeStruct((NP, D), f32),
    )(pp1, dinv, b1r, W2)

    pp2 = _sc_agg(hs2, src_p, dst_p, zeros_nodes)

    out_p = pl.pallas_call(
        _tc_last_body,
        out_shape=jax.ShapeDtypeStruct((NP, D), f32),
    )(pp2, dinv, b2r)

    return out_p[:N]


# speedup vs baseline: 33.6484x; 1.0194x over previous
import jax
import jax.numpy as jnp
from jax import lax
from jax.experimental import pallas as pl
from jax.experimental.pallas import tpu as pltpu
from jax.experimental.pallas import tpu_sc as plsc

N = 10000
NP = 10112
D = 128
E = 320000
K = 128
NTILES = 32
PH = 40
CH = 2 * PH
NCH = NTILES * CH
DCH = NCH // NTILES
E_PAD = NCH * K
RPT = NP // 16

_mesh = plsc.VectorSubcoreMesh(core_axis_name="c", subcore_axis_name="s")


def _sc_deg_body(dst_hbm, ones_hbm, zeros_hbm, out_hbm, idx_v, ones_v, tmp, acc):
    c = lax.axis_index("c")
    s = lax.axis_index("s")
    r0 = s * RPT

    @pl.when(c == 0)
    def _():
        pltpu.sync_copy(ones_hbm.at[pl.ds(r0, RPT)], tmp.at[pl.ds(0, RPT)])

    @pl.when(c != 0)
    def _():
        pltpu.sync_copy(zeros_hbm.at[pl.ds(r0, RPT)], tmp.at[pl.ds(0, RPT)])

    pltpu.sync_copy(tmp.at[pl.ds(0, RPT)], acc.at[pl.ds(r0, RPT)])

    pltpu.sync_copy(ones_hbm.at[pl.ds(0, K)], ones_v)
    wid = c * 16 + s
    pltpu.sync_copy(dst_hbm.at[pl.ds(wid * DCH, DCH)], idx_v)
    plsc.subcore_barrier()

    def body(i, carry):
        pltpu.sync_copy(ones_v, acc.at[idx_v.at[i]], add=True)
        return carry

    lax.fori_loop(0, DCH, body, 0)
    plsc.subcore_barrier()

    @pl.when(s == 0)
    def _():
        pltpu.sync_copy(acc, tmp)
        pltpu.sync_copy(tmp, out_hbm.at[c])


_sc_deg = pl.kernel(
    _sc_deg_body,
    out_type=jax.ShapeDtypeStruct((2, NP), jnp.float32),
    mesh=_mesh,
    scratch_types=[
        pltpu.VMEM((DCH, K), jnp.int32),
        pltpu.VMEM((K,), jnp.float32),
        pltpu.VMEM((NP,), jnp.float32),
        pltpu.VMEM_SHARED((NP,), jnp.float32),
    ],
)


def _sc_agg_body(hs_hbm, src_hbm, dst_hbm, zeros_hbm, out_hbm,
                 sidx, didx, rows0, rows1, acc, g0, g1):
    c = lax.axis_index("c")
    s = lax.axis_index("s")
    r0 = s * RPT

    @pl.when(c == 0)
    def _():
        pltpu.sync_copy(hs_hbm.at[pl.ds(r0, RPT)], acc.at[pl.ds(r0, RPT)])

    @pl.when(c != 0)
    def _():
        pltpu.sync_copy(zeros_hbm, acc.at[pl.ds(r0, RPT)])

    plsc.subcore_barrier()

    def run_phase(base):
        pltpu.sync_copy(src_hbm.at[pl.ds(base, PH)], sidx)
        pltpu.sync_copy(dst_hbm.at[pl.ds(base, PH)], didx)
        pltpu.async_copy(hs_hbm.at[sidx.at[0]], rows0, g0)
        pltpu.async_copy(hs_hbm.at[sidx.at[1]], rows1, g1)

        def body(j, carry):
            i0 = 2 * j
            i1 = 2 * j + 1
            pltpu.make_async_copy(hs_hbm.at[sidx.at[i0]], rows0, g0).wait()
            pltpu.sync_copy(rows0, acc.at[didx.at[i0]], add=True)

            @pl.when(i0 + 2 < PH)
            def _():
                pltpu.async_copy(hs_hbm.at[sidx.at[i0 + 2]], rows0, g0)

            pltpu.make_async_copy(hs_hbm.at[sidx.at[i1]], rows1, g1).wait()
            pltpu.sync_copy(rows1, acc.at[didx.at[i1]], add=True)

            @pl.when(i1 + 2 < PH)
            def _():
                pltpu.async_copy(hs_hbm.at[sidx.at[i1 + 2]], rows1, g1)

            return carry

        lax.fori_loop(0, PH // 2, body, 0)

    wid = c * 16 + s
    for p in range(CH // PH):
        run_phase(wid * CH + p * PH)

    plsc.subcore_barrier()
    pltpu.sync_copy(acc.at[pl.ds(r0, RPT)], out_hbm.at[c, pl.ds(r0, RPT)])


_sc_agg = pl.kernel(
    _sc_agg_body,
    out_type=jax.ShapeDtypeStruct((2, NP, D), jnp.float32),
    mesh=_mesh,
    scratch_types=[
        pltpu.VMEM((PH, K), jnp.int32),
        pltpu.VMEM((PH, K), jnp.int32),
        pltpu.VMEM((K, D), jnp.float32),
        pltpu.VMEM((K, D), jnp.float32),
        pltpu.VMEM_SHARED((NP, D), jnp.float32),
        pltpu.SemaphoreType.DMA,
        pltpu.SemaphoreType.DMA,
    ],
)


def _leaky(x):
    return jnp.where(x >= 0, x, 0.01 * x)


def _tc_first_body(x_ref, w_ref, deg_ref, hs_ref, dinv_ref):
    deg2 = deg_ref[...]
    deg = (deg2[0] + deg2[1])[:, None]
    row = lax.broadcasted_iota(jnp.int32, (NP, 1), 0)
    dinv = jnp.where((deg > 0) & (row < N), lax.rsqrt(deg), 0.0)
    h = jnp.dot(x_ref[...], w_ref[...], preferred_element_type=jnp.float32)
    hs_ref[pl.ds(0, N), :] = h * dinv[:N]
    hs_ref[pl.ds(N, NP - N), :] = jnp.zeros((NP - N, D), jnp.float32)
    dinv_ref[...] = dinv


def _tc_mid_body(pp_ref, dinv_ref, b_ref, w_ref, hs_ref):
    pp = pp_ref[...]
    dinv = dinv_ref[...]
    x = _leaky(dinv * (pp[0] + pp[1]) + b_ref[...])
    h = jnp.dot(x, w_ref[...], preferred_element_type=jnp.float32)
    hs_ref[...] = h * dinv


def _tc_last_body(pp_ref, dinv_ref, b_ref, out_ref):
    pp = pp_ref[...]
    dinv = dinv_ref[...]
    out_ref[...] = _leaky(dinv[:N] * (pp[0, :N] + pp[1, :N]) + b_ref[...])


def kernel(nodes_feature, edge_index, W1, b1, W2, b2):
    f32 = jnp.float32
    src = edge_index[0]
    dst = edge_index[1]
    npad = E_PAD - E
    pad_src = (jnp.arange(npad, dtype=jnp.int32) * 131) % N
    pad_dst = N + (jnp.arange(npad, dtype=jnp.int32) % (NP - N))
    src_p = jnp.concatenate([src, pad_src]).reshape(NCH, K)
    dst_p = jnp.concatenate([dst, pad_dst]).reshape(NCH, K)
    zeros_nodes = jnp.zeros((RPT, D), f32)
    ones_deg = jnp.ones((NP,), f32)
    zeros_deg = jnp.zeros((NP,), f32)
    b1r = b1.reshape(1, D)
    b2r = b2.reshape(1, D)

    deg2 = _sc_deg(dst_p, ones_deg, zeros_deg)

    hs1, dinv = pl.pallas_call(
        _tc_first_body,
        out_shape=[
            jax.ShapeDtypeStruct((NP, D), f32),
            jax.ShapeDtypeStruct((NP, 1), f32),
        ],
    )(nodes_feature, W1, deg2)

    pp1 = _sc_agg(hs1, src_p, dst_p, zeros_nodes)

    hs2 = pl.pallas_call(
        _tc_mid_body,
        out_shape=jax.ShapeDtypeStruct((NP, D), f32),
    )(pp1, dinv, b1r, W2)

    pp2 = _sc_agg(hs2, src_p, dst_p, zeros_nodes)

    return pl.pallas_call(
        _tc_last_body,
        out_shape=jax.ShapeDtypeStruct((N, D), f32),
    )(pp2, dinv, b2r)
